# baseline (speedup 1.0000x reference)
import functools
import math

import jax
import jax.numpy as jnp
from jax import lax
from jax.experimental import pallas as pl
from jax.experimental.pallas import tpu as pltpu
from jax.experimental.pallas import tpu_sc as plsc

N = 10000
H = 128
NPAD = 10240
NC = 2
NS = 16
NW = NC * NS
CH = 128
CHA = 64
NBUF = 4
RPT = NPAD // NS
BN_EPS = 1e-5
_BN_SCALE = 1.0 / math.sqrt(1.0 + BN_EPS)

_MESH = plsc.VectorSubcoreMesh(core_axis_name="c", subcore_axis_name="s",
                               num_cores=NC, num_subcores=NS)


def _worker_ids():
    c = lax.axis_index("c")
    s = lax.axis_index("s")
    return c, s, s * NC + c


def _deg_body(nch, dst_hbm, degp_hbm, deg_sh, dst_v, ones_v, sem):
    c, s, wid = _worker_ids()
    hc = nch // 2

    @pl.loop(0, CH)
    def _zfill(i):
        for q in range(H // 16):
            ones_v[i, pl.ds(q * 16, 16)] = jnp.zeros((16,), jnp.float32)

    @pl.loop(0, RPT // CH)
    def _zero(r):
        pltpu.sync_copy(ones_v, deg_sh.at[pl.ds(s * RPT + r * CH, CH)])

    @pl.loop(0, CH)
    def _ofill(i):
        for q in range(H // 16):
            ones_v[i, pl.ds(q * 16, 16)] = jnp.ones((16,), jnp.float32)

    plsc.subcore_barrier()

    for half in range(2):
        pltpu.sync_copy(dst_hbm.at[pl.ds(wid * nch + half * hc, hc)], dst_v)

        @pl.loop(0, hc)
        def _scat(j):
            pltpu.async_copy(ones_v, deg_sh.at[dst_v.at[j]], sem, add=True)

            @pl.when(j >= 8)
            def _w():
                pltpu.make_async_copy(ones_v, deg_sh.at[dst_v.at[0]],
                                      sem).wait()

        @pl.loop(0, 8)
        def _tail(j):
            pltpu.make_async_copy(ones_v, deg_sh.at[dst_v.at[0]], sem).wait()

    plsc.subcore_barrier()
    _drain(deg_sh, degp_hbm, c, s)


def _drain(sh, out_hbm, c, s):
    last = N - (NS - 1) * RPT

    @pl.when(s < NS - 1)
    def _full():
        pltpu.sync_copy(sh.at[pl.ds(s * RPT, RPT)],
                        out_hbm.at[c, pl.ds(s * RPT, RPT)])

    @pl.when(s == NS - 1)
    def _clip():
        pltpu.sync_copy(sh.at[pl.ds((NS - 1) * RPT, last)],
                        out_hbm.at[c, pl.ds((NS - 1) * RPT, last)])


def _deg_call(dst2, nch):
    return pl.kernel(
        functools.partial(_deg_body, nch),
        out_type=jax.ShapeDtypeStruct((NC, N, H), jnp.float32),
        mesh=_MESH,
        scratch_types=[
            pltpu.VMEM_SHARED((NPAD, H), jnp.float32),
            pltpu.VMEM((nch // 2, CH), jnp.int32),
            pltpu.VMEM((CH, H), jnp.float32),
            pltpu.SemaphoreType.DMA,
        ],
    )(dst2)


def _agg_body(nch, g_hbm, src_hbm, dst_hbm, accp_hbm, acc_sh, src_v, dst_v,
              rows, g0, g1, g2, g3, s0, s1, s2, s3):
    c, s, wid = _worker_ids()
    hc = nch // 4
    gsem = (g0, g1, g2, g3)
    ssem = (s0, s1, s2, s3)

    def wait_g(b):
        pltpu.make_async_copy(g_hbm.at[src_v.at[0]], rows.at[b],
                              gsem[b]).wait()

    def wait_s(b):
        pltpu.make_async_copy(rows.at[b], acc_sh.at[dst_v.at[0]],
                              ssem[b]).wait()

    @pl.loop(0, CHA)
    def _zfill(i):
        for q in range(H // 16):
            rows[0, i, pl.ds(q * 16, 16)] = jnp.zeros((16,), jnp.float32)

    @pl.loop(0, RPT // CHA)
    def _zero(r):
        pltpu.sync_copy(rows.at[0],
                        acc_sh.at[pl.ds(s * RPT + r * CHA, CHA)])

    plsc.subcore_barrier()

    for stage in range(4):
        base = wid * nch + stage * hc
        pltpu.sync_copy(src_hbm.at[pl.ds(base, hc)], src_v)
        pltpu.sync_copy(dst_hbm.at[pl.ds(base, hc)], dst_v)

        for b in range(2):
            pltpu.async_copy(g_hbm.at[src_v.at[b]], rows.at[b], gsem[b])

        @pl.loop(0, hc // 4)
        def _steps(t):
            j4 = t * 4
            for b in range(4):
                m = j4 + b
                b2 = (b + 2) % 4
                wait_g(b)
                pltpu.async_copy(rows.at[b], acc_sh.at[dst_v.at[m]],
                                 ssem[b], add=True)
                if b < 2:
                    @pl.when(t > 0)
                    def _w():
                        wait_s(b2)
                else:
                    wait_s(b2)

                @pl.when(m + 2 < hc)
                def _pf():
                    pltpu.async_copy(g_hbm.at[src_v.at[m + 2]], rows.at[b2],
                                     gsem[b2])

        wait_s(2)
        wait_s(3)

    plsc.subcore_barrier()
    _drain(acc_sh, accp_hbm, c, s)


def _agg_call(g, src2, dst2, nch):
    return pl.kernel(
        functools.partial(_agg_body, nch),
        out_type=jax.ShapeDtypeStruct((NC, N, H), jnp.float32),
        mesh=_MESH,
        scratch_types=[
            pltpu.VMEM_SHARED((NPAD, H), jnp.float32),
            pltpu.VMEM((nch // 4, CHA), jnp.int32),
            pltpu.VMEM((nch // 4, CHA), jnp.int32),
            pltpu.VMEM((NBUF, CHA, H), jnp.float32),
        ] + [pltpu.SemaphoreType.DMA] * 8,
    )(g, src2, dst2)


BM = 2000


def _dinv_from(degp_ref):
    deg = 1.0 + degp_ref[0, :, 0:1] + degp_ref[1, :, 0:1]
    return lax.rsqrt(deg)


def _mm_t(x, w_ref):
    return lax.dot_general(x, w_ref[:], (((1,), (1,)), ((), ())),
                           preferred_element_type=jnp.float32)


def _pre_body(idw, n2v, win, bin_, w0, degp, x0_o, h0_o, g0_o):
    xa = lax.dot_general(idw[:], win[:, :H], (((1,), (1,)), ((), ())),
                         preferred_element_type=jnp.float32)
    xb = lax.dot_general(n2v[:], win[:, H:], (((1,), (1,)), ((), ())),
                         preferred_element_type=jnp.float32)
    x0 = xa + xb + bin_[:]
    h0 = _mm_t(x0, w0)
    dinv = _dinv_from(degp)
    x0_o[:] = x0
    h0_o[:] = h0
    g0_o[:] = dinv * h0


def _layer_body(xp, hp, accp, degp, bc, bng, bnb, wn, xo, ho, go):
    dinv = _dinv_from(degp)
    agg = accp[0] + accp[1]
    conv = dinv * agg + (dinv * dinv) * hp[:] + bc[:]
    xbn = conv * (bng[:] * _BN_SCALE) + bnb[:]
    xn = xp[:] + jnp.maximum(xbn, 0.0)
    xo[:] = xn
    hn = _mm_t(xn, wn)
    ho[:] = hn
    go[:] = dinv * hn


def _final_body(xp, hp, accp, degp, bc, bng, bnb, xo):
    dinv = _dinv_from(degp)
    agg = accp[0] + accp[1]
    conv = dinv * agg + (dinv * dinv) * hp[:] + bc[:]
    xbn = conv * (bng[:] * _BN_SCALE) + bnb[:]
    xo[:] = xp[:] + jnp.maximum(xbn, 0.0)


_F32 = lambda *s: jax.ShapeDtypeStruct(s, jnp.float32)

_ROWS = pl.BlockSpec((BM, H), lambda i: (i, 0))
_CONST = lambda *shape: pl.BlockSpec(shape, lambda i: (0,) * len(shape))
_PARTS = pl.BlockSpec((NC, BM, H), lambda i: (0, i, 0))
_DEGS = pl.BlockSpec((NC, BM, H), lambda i: (0, i, 0))


def kernel(edge_index, id_emb_w, n2v_w, W_in, b_in, convW0, convb0, bn_g0,
           bn_b0, convW1, convb1, bn_g1, bn_b1):
    src, dst = edge_index[0], edge_index[1]
    E = src.shape[0]
    nch = -(-E // (NW * CH))
    nch = -(-nch // 8) * 8
    epad = NW * nch * CH
    pad = epad - E
    src_p = jnp.concatenate([src, jnp.zeros((pad,), jnp.int32)])
    dst_p = jnp.concatenate([dst, jnp.full((pad,), NPAD - 1, jnp.int32)])
    dst2 = dst_p.reshape(NW * nch, CH)
    ncha = nch * (CH // CHA)
    src2a = src_p.reshape(NW * ncha, CHA)
    dst2a = dst_p.reshape(NW * ncha, CHA)

    degp = _deg_call(dst2, nch)

    b_in2 = b_in.reshape(1, H)
    x0, h0, g0 = pl.pallas_call(
        _pre_body,
        grid=(N // BM,),
        in_specs=[_ROWS, _ROWS, _CONST(H, 2 * H), _CONST(1, H),
                  _CONST(H, H), _DEGS],
        out_specs=[_ROWS, _ROWS, _ROWS],
        out_shape=[_F32(N, H), _F32(N, H), _F32(N, H)],
    )(id_emb_w, n2v_w, W_in, b_in2, convW0, degp)

    accp0 = _agg_call(g0, src2a, dst2a, ncha)

    x1, h1, g1 = pl.pallas_call(
        _layer_body,
        grid=(N // BM,),
        in_specs=[_ROWS, _ROWS, _PARTS, _DEGS, _CONST(1, H), _CONST(1, H),
                  _CONST(1, H), _CONST(H, H)],
        out_specs=[_ROWS, _ROWS, _ROWS],
        out_shape=[_F32(N, H), _F32(N, H), _F32(N, H)],
    )(x0, h0, accp0, degp, convb0.reshape(1, H), bn_g0.reshape(1, H),
      bn_b0.reshape(1, H), convW1)

    accp1 = _agg_call(g1, src2a, dst2a, ncha)

    out = pl.pallas_call(
        _final_body,
        grid=(N // BM,),
        in_specs=[_ROWS, _ROWS, _PARTS, _DEGS, _CONST(1, H), _CONST(1, H),
                  _CONST(1, H)],
        out_specs=_ROWS,
        out_shape=_F32(N, H),
    )(x1, h1, accp1, degp, convb1.reshape(1, H), bn_g1.reshape(1, H),
      bn_b1.reshape(1, H))
    return out


# speedup vs baseline: 1.1903x; 1.1903x over previous
import functools
import math

import jax
import jax.numpy as jnp
from jax import lax
from jax.experimental import pallas as pl
from jax.experimental.pallas import tpu as pltpu
from jax.experimental.pallas import tpu_sc as plsc

N = 10000
H = 128
NPAD = 10240
NC = 2
NS = 16
NW = NC * NS
CH = 128
RPT = NPAD // NS
BN_EPS = 1e-5
_BN_SCALE = 1.0 / math.sqrt(1.0 + BN_EPS)

_MESH = plsc.VectorSubcoreMesh(core_axis_name="c", subcore_axis_name="s",
                               num_cores=NC, num_subcores=NS)


def _worker_ids():
    c = lax.axis_index("c")
    s = lax.axis_index("s")
    return c, s, s * NC + c


def _deg_body(nch, dst_hbm, degp_hbm, deg_sh, dst_v, ones_v, sem):
    c, s, wid = _worker_ids()
    hc = nch // 2

    @pl.loop(0, CH)
    def _zfill(i):
        for q in range(H // 16):
            ones_v[i, pl.ds(q * 16, 16)] = jnp.zeros((16,), jnp.float32)

    @pl.loop(0, RPT // CH)
    def _zero(r):
        pltpu.sync_copy(ones_v, deg_sh.at[pl.ds(s * RPT + r * CH, CH)])

    @pl.loop(0, CH)
    def _ofill(i):
        for q in range(H // 16):
            ones_v[i, pl.ds(q * 16, 16)] = jnp.ones((16,), jnp.float32)

    plsc.subcore_barrier()

    for half in range(2):
        pltpu.sync_copy(dst_hbm.at[pl.ds(wid * nch + half * hc, hc)], dst_v)

        @pl.loop(0, hc)
        def _scat(j):
            pltpu.async_copy(ones_v, deg_sh.at[dst_v.at[j]], sem, add=True)

            @pl.when(j >= 8)
            def _w():
                pltpu.make_async_copy(ones_v, deg_sh.at[dst_v.at[0]],
                                      sem).wait()

        @pl.loop(0, 8)
        def _tail(j):
            pltpu.make_async_copy(ones_v, deg_sh.at[dst_v.at[0]], sem).wait()

    plsc.subcore_barrier()
    _drain(deg_sh, degp_hbm, c, s)


def _drain(sh, out_hbm, c, s):
    last = N - (NS - 1) * RPT

    @pl.when(s < NS - 1)
    def _full():
        pltpu.sync_copy(sh.at[pl.ds(s * RPT, RPT)],
                        out_hbm.at[c, pl.ds(s * RPT, RPT)])

    @pl.when(s == NS - 1)
    def _clip():
        pltpu.sync_copy(sh.at[pl.ds((NS - 1) * RPT, last)],
                        out_hbm.at[c, pl.ds((NS - 1) * RPT, last)])


def _deg_call(dst2, nch):
    return pl.kernel(
        functools.partial(_deg_body, nch),
        out_type=jax.ShapeDtypeStruct((NC, N, H), jnp.float32),
        mesh=_MESH,
        scratch_types=[
            pltpu.VMEM_SHARED((NPAD, H), jnp.float32),
            pltpu.VMEM((nch // 2, CH), jnp.int32),
            pltpu.VMEM((CH, H), jnp.float32),
            pltpu.SemaphoreType.DMA,
        ],
    )(dst2)


def _agg_body(nch, g_hbm, src_hbm, dst_hbm, accp_hbm, acc_sh, src_v, dst_v,
              rows, g0, g1, s0, s1):
    c, s, wid = _worker_ids()
    hc = nch // 2
    gsem = (g0, g1)
    ssem = (s0, s1)

    def wait_g(b):
        pltpu.make_async_copy(g_hbm.at[src_v.at[0]], rows.at[b],
                              gsem[b]).wait()

    def wait_s(b):
        pltpu.make_async_copy(rows.at[b], acc_sh.at[dst_v.at[0]],
                              ssem[b]).wait()

    @pl.loop(0, CH)
    def _zfill(i):
        for q in range(H // 16):
            rows[0, i, pl.ds(q * 16, 16)] = jnp.zeros((16,), jnp.float32)

    @pl.loop(0, RPT // CH)
    def _zero(r):
        pltpu.sync_copy(rows.at[0],
                        acc_sh.at[pl.ds(s * RPT + r * CH, CH)])

    plsc.subcore_barrier()

    for stage in range(2):
        base = wid * nch + stage * hc
        pltpu.sync_copy(src_hbm.at[pl.ds(base, hc)], src_v)
        pltpu.sync_copy(dst_hbm.at[pl.ds(base, hc)], dst_v)

        pltpu.async_copy(g_hbm.at[src_v.at[0]], rows.at[0], gsem[0])

        @pl.loop(0, hc // 2)
        def _steps(t):
            for b in range(2):
                m = t * 2 + b
                wait_g(b)
                pltpu.async_copy(rows.at[b], acc_sh.at[dst_v.at[m]],
                                 ssem[b], add=True)
                if b == 0:
                    @pl.when(t > 0)
                    def _w0():
                        wait_s(1)

                    pltpu.async_copy(g_hbm.at[src_v.at[m + 1]], rows.at[1],
                                     gsem[1])
                else:
                    wait_s(0)

                    @pl.when(t + 1 < hc // 2)
                    def _pf1():
                        pltpu.async_copy(g_hbm.at[src_v.at[m + 1]],
                                         rows.at[0], gsem[0])

        wait_s(1)

    plsc.subcore_barrier()
    _drain(acc_sh, accp_hbm, c, s)


def _agg_call(g, src2, dst2, nch):
    return pl.kernel(
        functools.partial(_agg_body, nch),
        out_type=jax.ShapeDtypeStruct((NC, N, H), jnp.float32),
        mesh=_MESH,
        scratch_types=[
            pltpu.VMEM_SHARED((NPAD, H), jnp.float32),
            pltpu.VMEM((nch // 2, CH), jnp.int32),
            pltpu.VMEM((nch // 2, CH), jnp.int32),
            pltpu.VMEM((2, CH, H), jnp.float32),
        ] + [pltpu.SemaphoreType.DMA] * 4,
    )(g, src2, dst2)


BM = 2000


def _dinv_from(degp_ref):
    deg = 1.0 + degp_ref[0, :, 0:1] + degp_ref[1, :, 0:1]
    return lax.rsqrt(deg)


def _mm_t(x, w_ref):
    return lax.dot_general(x, w_ref[:], (((1,), (1,)), ((), ())),
                           preferred_element_type=jnp.float32)


def _pre_body(idw, n2v, win, bin_, w0, degp, x0_o, h0_o, g0_o):
    xa = lax.dot_general(idw[:], win[:, :H], (((1,), (1,)), ((), ())),
                         preferred_element_type=jnp.float32)
    xb = lax.dot_general(n2v[:], win[:, H:], (((1,), (1,)), ((), ())),
                         preferred_element_type=jnp.float32)
    x0 = xa + xb + bin_[:]
    h0 = _mm_t(x0, w0)
    dinv = _dinv_from(degp)
    x0_o[:] = x0
    h0_o[:] = h0
    g0_o[:] = dinv * h0


def _layer_body(xp, hp, accp, degp, bc, bng, bnb, wn, xo, ho, go):
    dinv = _dinv_from(degp)
    agg = accp[0] + accp[1]
    conv = dinv * agg + (dinv * dinv) * hp[:] + bc[:]
    xbn = conv * (bng[:] * _BN_SCALE) + bnb[:]
    xn = xp[:] + jnp.maximum(xbn, 0.0)
    xo[:] = xn
    hn = _mm_t(xn, wn)
    ho[:] = hn
    go[:] = dinv * hn


def _final_body(xp, hp, accp, degp, bc, bng, bnb, xo):
    dinv = _dinv_from(degp)
    agg = accp[0] + accp[1]
    conv = dinv * agg + (dinv * dinv) * hp[:] + bc[:]
    xbn = conv * (bng[:] * _BN_SCALE) + bnb[:]
    xo[:] = xp[:] + jnp.maximum(xbn, 0.0)


_F32 = lambda *s: jax.ShapeDtypeStruct(s, jnp.float32)

_ROWS = pl.BlockSpec((BM, H), lambda i: (i, 0))
_CONST = lambda *shape: pl.BlockSpec(shape, lambda i: (0,) * len(shape))
_PARTS = pl.BlockSpec((NC, BM, H), lambda i: (0, i, 0))
_DEGS = pl.BlockSpec((NC, BM, H), lambda i: (0, i, 0))


def kernel(edge_index, id_emb_w, n2v_w, W_in, b_in, convW0, convb0, bn_g0,
           bn_b0, convW1, convb1, bn_g1, bn_b1):
    src, dst = edge_index[0], edge_index[1]
    E = src.shape[0]
    nch = -(-E // (NW * CH))
    nch = -(-nch // 8) * 8
    epad = NW * nch * CH
    pad = epad - E
    src_p = jnp.concatenate([src, jnp.zeros((pad,), jnp.int32)])
    dst_p = jnp.concatenate([dst, jnp.full((pad,), NPAD - 1, jnp.int32)])
    src2 = src_p.reshape(NW * nch, CH)
    dst2 = dst_p.reshape(NW * nch, CH)

    degp = _deg_call(dst2, nch)

    b_in2 = b_in.reshape(1, H)
    x0, h0, g0 = pl.pallas_call(
        _pre_body,
        grid=(N // BM,),
        in_specs=[_ROWS, _ROWS, _CONST(H, 2 * H), _CONST(1, H),
                  _CONST(H, H), _DEGS],
        out_specs=[_ROWS, _ROWS, _ROWS],
        out_shape=[_F32(N, H), _F32(N, H), _F32(N, H)],
    )(id_emb_w, n2v_w, W_in, b_in2, convW0, degp)

    accp0 = _agg_call(g0, src2, dst2, nch)

    x1, h1, g1 = pl.pallas_call(
        _layer_body,
        grid=(N // BM,),
        in_specs=[_ROWS, _ROWS, _PARTS, _DEGS, _CONST(1, H), _CONST(1, H),
                  _CONST(1, H), _CONST(H, H)],
        out_specs=[_ROWS, _ROWS, _ROWS],
        out_shape=[_F32(N, H), _F32(N, H), _F32(N, H)],
    )(x0, h0, accp0, degp, convb0.reshape(1, H), bn_g0.reshape(1, H),
      bn_b0.reshape(1, H), convW1)

    accp1 = _agg_call(g1, src2, dst2, nch)

    out = pl.pallas_call(
        _final_body,
        grid=(N // BM,),
        in_specs=[_ROWS, _ROWS, _PARTS, _DEGS, _CONST(1, H), _CONST(1, H),
                  _CONST(1, H)],
        out_specs=_ROWS,
        out_shape=_F32(N, H),
    )(x1, h1, accp1, degp, convb1.reshape(1, H), bn_g1.reshape(1, H),
      bn_b1.reshape(1, H))
    return out


# speedup vs baseline: 1.1909x; 1.0005x over previous
import functools
import math

import jax
import jax.numpy as jnp
from jax import lax
from jax.experimental import pallas as pl
from jax.experimental.pallas import tpu as pltpu
from jax.experimental.pallas import tpu_sc as plsc

N = 10000
H = 128
NPAD = 10240
NC = 2
NS = 16
NW = NC * NS
CH = 128
RPT = NPAD // NS
BN_EPS = 1e-5
_BN_SCALE = 1.0 / math.sqrt(1.0 + BN_EPS)

_MESH = plsc.VectorSubcoreMesh(core_axis_name="c", subcore_axis_name="s",
                               num_cores=NC, num_subcores=NS)


def _worker_ids():
    c = lax.axis_index("c")
    s = lax.axis_index("s")
    return c, s, s * NC + c


def _deg_body(nch, dst_hbm, degp_hbm, deg_sh, dst_v, ones_v, sem):
    c, s, wid = _worker_ids()
    hc = nch // 2

    @pl.loop(0, CH)
    def _zfill(i):
        for q in range(H // 16):
            ones_v[i, pl.ds(q * 16, 16)] = jnp.zeros((16,), jnp.float32)

    @pl.loop(0, RPT // CH)
    def _zero(r):
        pltpu.sync_copy(ones_v, deg_sh.at[pl.ds(s * RPT + r * CH, CH)])

    @pl.loop(0, CH)
    def _ofill(i):
        for q in range(H // 16):
            ones_v[i, pl.ds(q * 16, 16)] = jnp.ones((16,), jnp.float32)

    plsc.subcore_barrier()

    for half in range(2):
        pltpu.sync_copy(dst_hbm.at[pl.ds(wid * nch + half * hc, hc)], dst_v)

        @pl.loop(0, hc)
        def _scat(j):
            pltpu.async_copy(ones_v, deg_sh.at[dst_v.at[j]], sem, add=True)

            @pl.when(j >= 8)
            def _w():
                pltpu.make_async_copy(ones_v, deg_sh.at[dst_v.at[0]],
                                      sem).wait()

        @pl.loop(0, 8)
        def _tail(j):
            pltpu.make_async_copy(ones_v, deg_sh.at[dst_v.at[0]], sem).wait()

    plsc.subcore_barrier()
    _drain(deg_sh, degp_hbm, c, s)


def _drain(sh, out_hbm, c, s):
    last = N - (NS - 1) * RPT

    @pl.when(s < NS - 1)
    def _full():
        pltpu.sync_copy(sh.at[pl.ds(s * RPT, RPT)],
                        out_hbm.at[c, pl.ds(s * RPT, RPT)])

    @pl.when(s == NS - 1)
    def _clip():
        pltpu.sync_copy(sh.at[pl.ds((NS - 1) * RPT, last)],
                        out_hbm.at[c, pl.ds((NS - 1) * RPT, last)])


def _deg_call(dst2, nch):
    return pl.kernel(
        functools.partial(_deg_body, nch),
        out_type=jax.ShapeDtypeStruct((NC, N, H), jnp.float32),
        mesh=_MESH,
        scratch_types=[
            pltpu.VMEM_SHARED((NPAD, H), jnp.float32),
            pltpu.VMEM((nch // 2, CH), jnp.int32),
            pltpu.VMEM((CH, H), jnp.float32),
            pltpu.SemaphoreType.DMA,
        ],
    )(dst2)


def _agg_body(nch, g_hbm, src_hbm, dst_hbm, accp_hbm, acc_sh, src_v, dst_v,
              rows, g0, g1, g2, g3, s0, s1):
    c, s, wid = _worker_ids()
    hc = nch // 2
    gsem = ((g0, g1), (g2, g3))
    ssem = (s0, s1)
    HCH = CH // 2

    def fire_g(m, b):
        pltpu.async_copy(g_hbm.at[src_v.at[m, pl.ds(0, HCH)]],
                         rows.at[b, pl.ds(0, HCH)], gsem[b][0])
        pltpu.async_copy(g_hbm.at[src_v.at[m, pl.ds(HCH, HCH)]],
                         rows.at[b, pl.ds(HCH, HCH)], gsem[b][1])

    def wait_g(b):
        pltpu.make_async_copy(g_hbm.at[src_v.at[0, pl.ds(0, HCH)]],
                              rows.at[b, pl.ds(0, HCH)], gsem[b][0]).wait()
        pltpu.make_async_copy(g_hbm.at[src_v.at[0, pl.ds(0, HCH)]],
                              rows.at[b, pl.ds(HCH, HCH)], gsem[b][1]).wait()

    def wait_s(b):
        pltpu.make_async_copy(rows.at[b], acc_sh.at[dst_v.at[0]],
                              ssem[b]).wait()

    @pl.loop(0, CH)
    def _zfill(i):
        for q in range(H // 16):
            rows[0, i, pl.ds(q * 16, 16)] = jnp.zeros((16,), jnp.float32)

    @pl.loop(0, RPT // CH)
    def _zero(r):
        pltpu.sync_copy(rows.at[0],
                        acc_sh.at[pl.ds(s * RPT + r * CH, CH)])

    plsc.subcore_barrier()

    for stage in range(2):
        base = wid * nch + stage * hc
        pltpu.sync_copy(src_hbm.at[pl.ds(base, hc)], src_v)
        pltpu.sync_copy(dst_hbm.at[pl.ds(base, hc)], dst_v)

        fire_g(0, 0)

        @pl.loop(0, hc // 2)
        def _steps(t):
            for b in range(2):
                m = t * 2 + b
                wait_g(b)
                pltpu.async_copy(rows.at[b], acc_sh.at[dst_v.at[m]],
                                 ssem[b], add=True)
                if b == 0:
                    @pl.when(t > 0)
                    def _w0():
                        wait_s(1)

                    fire_g(m + 1, 1)
                else:
                    wait_s(0)

                    @pl.when(t + 1 < hc // 2)
                    def _pf1():
                        fire_g(m + 1, 0)

        wait_s(1)

    plsc.subcore_barrier()
    _drain(acc_sh, accp_hbm, c, s)


def _agg_call(g, src2, dst2, nch):
    return pl.kernel(
        functools.partial(_agg_body, nch),
        out_type=jax.ShapeDtypeStruct((NC, N, H), jnp.float32),
        mesh=_MESH,
        scratch_types=[
            pltpu.VMEM_SHARED((NPAD, H), jnp.float32),
            pltpu.VMEM((nch // 2, CH), jnp.int32),
            pltpu.VMEM((nch // 2, CH), jnp.int32),
            pltpu.VMEM((2, CH, H), jnp.float32),
        ] + [pltpu.SemaphoreType.DMA] * 6,
    )(g, src2, dst2)


BM = 2000


def _dinv_from(degp_ref):
    deg = 1.0 + degp_ref[0, :, 0:1] + degp_ref[1, :, 0:1]
    return lax.rsqrt(deg)


def _mm_t(x, w_ref):
    return lax.dot_general(x, w_ref[:], (((1,), (1,)), ((), ())),
                           preferred_element_type=jnp.float32)


def _pre_body(idw, n2v, win, bin_, w0, degp, x0_o, h0_o, g0_o):
    xa = lax.dot_general(idw[:], win[:, :H], (((1,), (1,)), ((), ())),
                         preferred_element_type=jnp.float32)
    xb = lax.dot_general(n2v[:], win[:, H:], (((1,), (1,)), ((), ())),
                         preferred_element_type=jnp.float32)
    x0 = xa + xb + bin_[:]
    h0 = _mm_t(x0, w0)
    dinv = _dinv_from(degp)
    x0_o[:] = x0
    h0_o[:] = h0
    g0_o[:] = dinv * h0


def _layer_body(xp, hp, accp, degp, bc, bng, bnb, wn, xo, ho, go):
    dinv = _dinv_from(degp)
    agg = accp[0] + accp[1]
    conv = dinv * agg + (dinv * dinv) * hp[:] + bc[:]
    xbn = conv * (bng[:] * _BN_SCALE) + bnb[:]
    xn = xp[:] + jnp.maximum(xbn, 0.0)
    xo[:] = xn
    hn = _mm_t(xn, wn)
    ho[:] = hn
    go[:] = dinv * hn


def _final_body(xp, hp, accp, degp, bc, bng, bnb, xo):
    dinv = _dinv_from(degp)
    agg = accp[0] + accp[1]
    conv = dinv * agg + (dinv * dinv) * hp[:] + bc[:]
    xbn = conv * (bng[:] * _BN_SCALE) + bnb[:]
    xo[:] = xp[:] + jnp.maximum(xbn, 0.0)


_F32 = lambda *s: jax.ShapeDtypeStruct(s, jnp.float32)

_ROWS = pl.BlockSpec((BM, H), lambda i: (i, 0))
_CONST = lambda *shape: pl.BlockSpec(shape, lambda i: (0,) * len(shape))
_PARTS = pl.BlockSpec((NC, BM, H), lambda i: (0, i, 0))
_DEGS = pl.BlockSpec((NC, BM, H), lambda i: (0, i, 0))


def kernel(edge_index, id_emb_w, n2v_w, W_in, b_in, convW0, convb0, bn_g0,
           bn_b0, convW1, convb1, bn_g1, bn_b1):
    src, dst = edge_index[0], edge_index[1]
    E = src.shape[0]
    nch = -(-E // (NW * CH))
    nch = -(-nch // 8) * 8
    epad = NW * nch * CH
    pad = epad - E
    src_p = jnp.concatenate([src, jnp.zeros((pad,), jnp.int32)])
    dump = N + jnp.arange(pad, dtype=jnp.int32) % (NPAD - N)
    dst_p = jnp.concatenate([dst, dump])
    src2 = src_p.reshape(NW * nch, CH)
    dst2 = dst_p.reshape(NW * nch, CH)

    degp = _deg_call(dst2, nch)

    b_in2 = b_in.reshape(1, H)
    x0, h0, g0 = pl.pallas_call(
        _pre_body,
        grid=(N // BM,),
        in_specs=[_ROWS, _ROWS, _CONST(H, 2 * H), _CONST(1, H),
                  _CONST(H, H), _DEGS],
        out_specs=[_ROWS, _ROWS, _ROWS],
        out_shape=[_F32(N, H), _F32(N, H), _F32(N, H)],
    )(id_emb_w, n2v_w, W_in, b_in2, convW0, degp)

    accp0 = _agg_call(g0, src2, dst2, nch)

    x1, h1, g1 = pl.pallas_call(
        _layer_body,
        grid=(N // BM,),
        in_specs=[_ROWS, _ROWS, _PARTS, _DEGS, _CONST(1, H), _CONST(1, H),
                  _CONST(1, H), _CONST(H, H)],
        out_specs=[_ROWS, _ROWS, _ROWS],
        out_shape=[_F32(N, H), _F32(N, H), _F32(N, H)],
    )(x0, h0, accp0, degp, convb0.reshape(1, H), bn_g0.reshape(1, H),
      bn_b0.reshape(1, H), convW1)

    accp1 = _agg_call(g1, src2, dst2, nch)

    out = pl.pallas_call(
        _final_body,
        grid=(N // BM,),
        in_specs=[_ROWS, _ROWS, _PARTS, _DEGS, _CONST(1, H), _CONST(1, H),
                  _CONST(1, H)],
        out_specs=_ROWS,
        out_shape=_F32(N, H),
    )(x1, h1, accp1, degp, convb1.reshape(1, H), bn_g1.reshape(1, H),
      bn_b1.reshape(1, H))
    return out


# speedup vs baseline: 1.3128x; 1.1024x over previous
import functools
import math

import jax
import jax.numpy as jnp
from jax import lax
from jax.experimental import pallas as pl
from jax.experimental.pallas import tpu as pltpu
from jax.experimental.pallas import tpu_sc as plsc

N = 10000
H = 128
NPAD = 10240
NC = 2
NS = 16
NW = NC * NS
CH = 128
RPT = NPAD // NS
FASTC = 0
FR = 112
BN_EPS = 1e-5
_BN_SCALE = 1.0 / math.sqrt(1.0 + BN_EPS)

_MESH = plsc.VectorSubcoreMesh(core_axis_name="c", subcore_axis_name="s",
                               num_cores=NC, num_subcores=NS)


def _worker_ids():
    c = lax.axis_index("c")
    s = lax.axis_index("s")
    return c, s, s * NC + c


def _deg_body(nch, dst_hbm, degp_hbm, deg_sh, dst_v, ones_v, sem):
    c, s, wid = _worker_ids()
    hc = nch // 2

    @pl.loop(0, CH)
    def _zfill(i):
        for q in range(H // 16):
            ones_v[i, pl.ds(q * 16, 16)] = jnp.zeros((16,), jnp.float32)

    @pl.loop(0, RPT // CH)
    def _zero(r):
        pltpu.sync_copy(ones_v, deg_sh.at[pl.ds(s * RPT + r * CH, CH)])

    @pl.loop(0, CH)
    def _ofill(i):
        for q in range(H // 16):
            ones_v[i, pl.ds(q * 16, 16)] = jnp.ones((16,), jnp.float32)

    plsc.subcore_barrier()

    for half in range(2):
        pltpu.sync_copy(dst_hbm.at[pl.ds(wid * nch + half * hc, hc)], dst_v)

        @pl.loop(0, hc)
        def _scat(j):
            pltpu.async_copy(ones_v, deg_sh.at[dst_v.at[j]], sem, add=True)

            @pl.when(j >= 8)
            def _w():
                pltpu.make_async_copy(ones_v, deg_sh.at[dst_v.at[0]],
                                      sem).wait()

        @pl.loop(0, 8)
        def _tail(j):
            pltpu.make_async_copy(ones_v, deg_sh.at[dst_v.at[0]], sem).wait()

    plsc.subcore_barrier()
    _drain(deg_sh, degp_hbm, c, s)


def _drain(sh, out_hbm, c, s):
    last = N - (NS - 1) * RPT

    @pl.when(s < NS - 1)
    def _full():
        pltpu.sync_copy(sh.at[pl.ds(s * RPT, RPT)],
                        out_hbm.at[c, pl.ds(s * RPT, RPT)])

    @pl.when(s == NS - 1)
    def _clip():
        pltpu.sync_copy(sh.at[pl.ds((NS - 1) * RPT, last)],
                        out_hbm.at[c, pl.ds((NS - 1) * RPT, last)])


def _deg_call(dst2, nch):
    return pl.kernel(
        functools.partial(_deg_body, nch),
        out_type=jax.ShapeDtypeStruct((NC, N, H), jnp.float32),
        mesh=_MESH,
        scratch_types=[
            pltpu.VMEM_SHARED((NPAD, H), jnp.float32),
            pltpu.VMEM((nch // 2, CH), jnp.int32),
            pltpu.VMEM((CH, H), jnp.float32),
            pltpu.SemaphoreType.DMA,
        ],
    )(dst2)


def _agg_body(nch, g_hbm, src_hbm, dst_hbm, accp_hbm, acc_sh, src_v, dst_v,
              rows, g0, g1, g2, g3, s0, s1):
    c, s, wid = _worker_ids()
    hc = nch // 2
    gsem = ((g0, g1), (g2, g3))
    ssem = (s0, s1)
    HCH = CH // 2

    def fire_g(m, b):
        pltpu.async_copy(g_hbm.at[src_v.at[m, pl.ds(0, HCH)]],
                         rows.at[b, pl.ds(0, HCH)], gsem[b][0])
        pltpu.async_copy(g_hbm.at[src_v.at[m, pl.ds(HCH, HCH)]],
                         rows.at[b, pl.ds(HCH, HCH)], gsem[b][1])

    def wait_g(b):
        pltpu.make_async_copy(g_hbm.at[src_v.at[0, pl.ds(0, HCH)]],
                              rows.at[b, pl.ds(0, HCH)], gsem[b][0]).wait()
        pltpu.make_async_copy(g_hbm.at[src_v.at[0, pl.ds(0, HCH)]],
                              rows.at[b, pl.ds(HCH, HCH)], gsem[b][1]).wait()

    def wait_s(b):
        pltpu.make_async_copy(rows.at[b], acc_sh.at[dst_v.at[0]],
                              ssem[b]).wait()

    @pl.loop(0, CH)
    def _zfill(i):
        for q in range(H // 16):
            rows[0, i, pl.ds(q * 16, 16)] = jnp.zeros((16,), jnp.float32)

    @pl.loop(0, RPT // CH)
    def _zero(r):
        pltpu.sync_copy(rows.at[0],
                        acc_sh.at[pl.ds(s * RPT + r * CH, CH)])

    plsc.subcore_barrier()

    def run(nck, base0):
        hcc = nck // 2
        for stage in range(2):
            base = base0 + stage * hcc
            pltpu.sync_copy(src_hbm.at[pl.ds(base, hcc)],
                            src_v.at[pl.ds(0, hcc)])
            pltpu.sync_copy(dst_hbm.at[pl.ds(base, hcc)],
                            dst_v.at[pl.ds(0, hcc)])

            fire_g(0, 0)

            @pl.loop(0, hcc // 2)
            def _steps(t):
                for b in range(2):
                    m = t * 2 + b
                    wait_g(b)
                    pltpu.async_copy(rows.at[b], acc_sh.at[dst_v.at[m]],
                                     ssem[b], add=True)
                    if b == 0:
                        @pl.when(t > 0)
                        def _w0():
                            wait_s(1)

                        fire_g(m + 1, 1)
                    else:
                        wait_s(0)

                        @pl.when(t + 1 < hcc // 2)
                        def _pf1():
                            fire_g(m + 1, 0)

            wait_s(1)

    blk = s * 2 * nch

    @pl.when(c == FASTC)
    def _fast():
        run(FR, blk)

    @pl.when(c == 1 - FASTC)
    def _slow():
        run(2 * nch - FR, blk + FR)

    plsc.subcore_barrier()
    _drain(acc_sh, accp_hbm, c, s)


def _agg_call(g, src2, dst2, nch):
    return pl.kernel(
        functools.partial(_agg_body, nch),
        out_type=jax.ShapeDtypeStruct((NC, N, H), jnp.float32),
        mesh=_MESH,
        scratch_types=[
            pltpu.VMEM_SHARED((NPAD, H), jnp.float32),
            pltpu.VMEM((FR // 2, CH), jnp.int32),
            pltpu.VMEM((FR // 2, CH), jnp.int32),
            pltpu.VMEM((2, CH, H), jnp.float32),
        ] + [pltpu.SemaphoreType.DMA] * 6,
    )(g, src2, dst2)


BM = 2000


def _dinv_from(degp_ref):
    deg = 1.0 + degp_ref[0, :, 0:1] + degp_ref[1, :, 0:1]
    return lax.rsqrt(deg)


def _mm_t(x, w_ref):
    return lax.dot_general(x, w_ref[:], (((1,), (1,)), ((), ())),
                           preferred_element_type=jnp.float32)


def _pre_body(idw, n2v, win, bin_, w0, degp, x0_o, h0_o, g0_o):
    xa = lax.dot_general(idw[:], win[:, :H], (((1,), (1,)), ((), ())),
                         preferred_element_type=jnp.float32)
    xb = lax.dot_general(n2v[:], win[:, H:], (((1,), (1,)), ((), ())),
                         preferred_element_type=jnp.float32)
    x0 = xa + xb + bin_[:]
    h0 = _mm_t(x0, w0)
    dinv = _dinv_from(degp)
    x0_o[:] = x0
    h0_o[:] = h0
    g0_o[:] = dinv * h0


def _layer_body(xp, hp, accp, degp, bc, bng, bnb, wn, xo, ho, go):
    dinv = _dinv_from(degp)
    agg = accp[0] + accp[1]
    conv = dinv * agg + (dinv * dinv) * hp[:] + bc[:]
    xbn = conv * (bng[:] * _BN_SCALE) + bnb[:]
    xn = xp[:] + jnp.maximum(xbn, 0.0)
    xo[:] = xn
    hn = _mm_t(xn, wn)
    ho[:] = hn
    go[:] = dinv * hn


def _final_body(xp, hp, accp, degp, bc, bng, bnb, xo):
    dinv = _dinv_from(degp)
    agg = accp[0] + accp[1]
    conv = dinv * agg + (dinv * dinv) * hp[:] + bc[:]
    xbn = conv * (bng[:] * _BN_SCALE) + bnb[:]
    xo[:] = xp[:] + jnp.maximum(xbn, 0.0)


_F32 = lambda *s: jax.ShapeDtypeStruct(s, jnp.float32)

_ROWS = pl.BlockSpec((BM, H), lambda i: (i, 0))
_CONST = lambda *shape: pl.BlockSpec(shape, lambda i: (0,) * len(shape))
_PARTS = pl.BlockSpec((NC, BM, H), lambda i: (0, i, 0))
_DEGS = pl.BlockSpec((NC, BM, H), lambda i: (0, i, 0))


def kernel(edge_index, id_emb_w, n2v_w, W_in, b_in, convW0, convb0, bn_g0,
           bn_b0, convW1, convb1, bn_g1, bn_b1):
    src, dst = edge_index[0], edge_index[1]
    E = src.shape[0]
    nch = -(-E // (NW * CH))
    nch = -(-nch // 8) * 8
    epad = NW * nch * CH
    pad = epad - E
    src_p = jnp.concatenate([src, jnp.zeros((pad,), jnp.int32)])
    dump = N + jnp.arange(pad, dtype=jnp.int32) % (NPAD - N)
    dst_p = jnp.concatenate([dst, dump])
    src2 = src_p.reshape(NW * nch, CH)
    dst2 = dst_p.reshape(NW * nch, CH)

    degp = _deg_call(dst2, nch)

    b_in2 = b_in.reshape(1, H)
    x0, h0, g0 = pl.pallas_call(
        _pre_body,
        grid=(N // BM,),
        in_specs=[_ROWS, _ROWS, _CONST(H, 2 * H), _CONST(1, H),
                  _CONST(H, H), _DEGS],
        out_specs=[_ROWS, _ROWS, _ROWS],
        out_shape=[_F32(N, H), _F32(N, H), _F32(N, H)],
    )(id_emb_w, n2v_w, W_in, b_in2, convW0, degp)

    accp0 = _agg_call(g0, src2, dst2, nch)

    x1, h1, g1 = pl.pallas_call(
        _layer_body,
        grid=(N // BM,),
        in_specs=[_ROWS, _ROWS, _PARTS, _DEGS, _CONST(1, H), _CONST(1, H),
                  _CONST(1, H), _CONST(H, H)],
        out_specs=[_ROWS, _ROWS, _ROWS],
        out_shape=[_F32(N, H), _F32(N, H), _F32(N, H)],
    )(x0, h0, accp0, degp, convb0.reshape(1, H), bn_g0.reshape(1, H),
      bn_b0.reshape(1, H), convW1)

    accp1 = _agg_call(g1, src2, dst2, nch)

    out = pl.pallas_call(
        _final_body,
        grid=(N // BM,),
        in_specs=[_ROWS, _ROWS, _PARTS, _DEGS, _CONST(1, H), _CONST(1, H),
                  _CONST(1, H)],
        out_specs=_ROWS,
        out_shape=_F32(N, H),
    )(x1, h1, accp1, degp, convb1.reshape(1, H), bn_g1.reshape(1, H),
      bn_b1.reshape(1, H))
    return out
